# Initial kernel scaffold; baseline (speedup 1.0000x reference)
import jax
import jax.numpy as jnp
from jax.experimental import pallas as pl


def kernel(node_feat, edge_index, edge_feat, W1, b1, W2, b2):
    raise NotImplementedError("write your pallas kernel here")


# speedup vs baseline: 12.6654x; 12.6654x over previous
import functools

import jax
import jax.numpy as jnp
from jax import lax
from jax.experimental import pallas as pl
from jax.experimental.pallas import tpu as pltpu
from jax.experimental.pallas import tpu_sc as plsc

N = 10000
E = 320000
D_IN = 128
D_HID = 16
D_OUT = 128

NC = 2
NS = 16
NW = NC * NS
CHUNK = 128
CPT = 79
EPT = CPT * CHUNK
EPAD = NW * EPT
NPAD = 10240
RPT = NPAD // NS
DEGW = 8

_mesh = plsc.VectorSubcoreMesh(core_axis_name="c", subcore_axis_name="s")


@functools.partial(
    pl.kernel,
    out_type=jax.ShapeDtypeStruct((NC, 2, NPAD, DEGW), jnp.float32),
    mesh=_mesh,
    scratch_types=[
        pltpu.VMEM((CPT, CHUNK), jnp.int32),
        pltpu.VMEM((CPT, CHUNK), jnp.int32),
        pltpu.VMEM((CHUNK, DEGW), jnp.float32),
        pltpu.VMEM_SHARED((NPAD, DEGW), jnp.float32),
        pltpu.VMEM_SHARED((NPAD, DEGW), jnp.float32),
    ],
    compiler_params=pltpu.CompilerParams(use_tc_tiling_on_sc=False),
)
def _sc_degrees(src_hbm, dst_hbm, ones_hbm, zeros_hbm, out_hbm,
                src_v, dst_v, ones_v, acc_src, acc_dst):
    cid = lax.axis_index("c")
    sid = lax.axis_index("s")
    wid = sid * NC + cid
    pltpu.sync_copy(src_hbm.at[wid], src_v)
    pltpu.sync_copy(dst_hbm.at[wid], dst_v)
    pltpu.sync_copy(ones_hbm, ones_v)
    rows = pl.ds(sid * RPT, RPT)
    pltpu.sync_copy(zeros_hbm.at[rows], acc_src.at[rows])
    pltpu.sync_copy(zeros_hbm.at[rows], acc_dst.at[rows])
    plsc.subcore_barrier()

    def body(j, carry):
        pltpu.sync_copy(ones_v, acc_src.at[src_v.at[j]], add=True)
        pltpu.sync_copy(ones_v, acc_dst.at[dst_v.at[j]], add=True)
        return carry

    lax.fori_loop(0, CPT, body, 0)
    plsc.subcore_barrier()
    pltpu.sync_copy(acc_src.at[rows], out_hbm.at[cid, 0, rows])
    pltpu.sync_copy(acc_dst.at[rows], out_hbm.at[cid, 1, rows])


@functools.partial(
    pl.kernel,
    out_type=jax.ShapeDtypeStruct((NC, NPAD, D_HID), jnp.float32),
    mesh=_mesh,
    scratch_types=[
        pltpu.VMEM((CPT, CHUNK), jnp.int32),
        pltpu.VMEM((CPT, CHUNK), jnp.int32),
        pltpu.VMEM((CHUNK, D_HID), jnp.float32),
        pltpu.VMEM_SHARED((NPAD, D_HID), jnp.float32),
        pltpu.SemaphoreType.DMA,
    ],
    compiler_params=pltpu.CompilerParams(use_tc_tiling_on_sc=False),
)
def _sc_aggregate(t_hbm, src_hbm, dst_hbm, zeros_hbm, out_hbm,
                  src_v, dst_v, msg_v, acc, sem):
    cid = lax.axis_index("c")
    sid = lax.axis_index("s")
    wid = sid * NC + cid
    pltpu.sync_copy(src_hbm.at[wid], src_v)
    pltpu.sync_copy(dst_hbm.at[wid], dst_v)
    rows = pl.ds(sid * RPT, RPT)
    pltpu.sync_copy(zeros_hbm.at[rows], acc.at[rows])
    plsc.subcore_barrier()

    def body(j, carry):
        pltpu.async_copy(t_hbm.at[src_v.at[j]], msg_v, sem).wait()
        pltpu.sync_copy(msg_v, acc.at[dst_v.at[j]], add=True)
        return carry

    lax.fori_loop(0, CPT, body, 0)
    plsc.subcore_barrier()
    pltpu.sync_copy(acc.at[rows], out_hbm.at[cid, rows])


def _tc_prep(nf_ref, w1_ref, degs_ref, t1_ref, rin_ref, rout_ref):
    d_out = degs_ref[0, 0, :, 0:1] + degs_ref[1, 0, :, 0:1]
    d_in = degs_ref[0, 1, :, 0:1] + degs_ref[1, 1, :, 0:1]
    rout = lax.rsqrt(jnp.maximum(d_out, 1.0))
    rin = lax.rsqrt(jnp.maximum(d_in, 1.0))
    p = jnp.dot(nf_ref[...], w1_ref[...], preferred_element_type=jnp.float32)
    t1_ref[...] = p * rout
    rin_ref[...] = jnp.broadcast_to(rin, (NPAD, D_HID))
    rout_ref[...] = jnp.broadcast_to(rout, (NPAD, D_HID))


def _tc_mid(agg_ref, rin_ref, rout_ref, b1_ref, t2_ref):
    s = agg_ref[0] + agg_ref[1]
    h = jnp.maximum(s * rin_ref[...] + b1_ref[...], 0.0)
    t2_ref[...] = h * rout_ref[...]


def _tc_out(agg_ref, rin_ref, w2_ref, b2_ref, out_ref):
    s = (agg_ref[0] + agg_ref[1]) * rin_ref[...]
    out_ref[...] = (
        jnp.dot(s, w2_ref[...], preferred_element_type=jnp.float32)
        + b2_ref[...]
    )


def kernel(node_feat, edge_index, edge_feat, W1, b1, W2, b2):
    del edge_feat

    src = edge_index[0]
    dst = edge_index[1]
    pad = jnp.full((EPAD - E,), N, jnp.int32)
    src_t = jnp.concatenate([src, pad]).reshape(NW, CPT, CHUNK)
    dst_t = jnp.concatenate([dst, pad]).reshape(NW, CPT, CHUNK)

    nf_pad = jnp.zeros((NPAD, D_IN), jnp.float32).at[:N].set(node_feat)
    ones_deg = jnp.ones((CHUNK, DEGW), jnp.float32)
    zeros_deg = jnp.zeros((NPAD, DEGW), jnp.float32)
    zeros_agg = jnp.zeros((NPAD, D_HID), jnp.float32)
    b1r = b1.reshape(1, D_HID)
    b2r = b2.reshape(1, D_OUT)

    degs = _sc_degrees(src_t, dst_t, ones_deg, zeros_deg)

    t1, rin, rout = pl.pallas_call(
        _tc_prep,
        out_shape=(
            jax.ShapeDtypeStruct((NPAD, D_HID), jnp.float32),
            jax.ShapeDtypeStruct((NPAD, D_HID), jnp.float32),
            jax.ShapeDtypeStruct((NPAD, D_HID), jnp.float32),
        ),
    )(nf_pad, W1, degs)

    agg1 = _sc_aggregate(t1, src_t, dst_t, zeros_agg)

    t2 = pl.pallas_call(
        _tc_mid,
        out_shape=jax.ShapeDtypeStruct((NPAD, D_HID), jnp.float32),
    )(agg1, rin, rout, b1r)

    agg2 = _sc_aggregate(t2, src_t, dst_t, zeros_agg)

    out = pl.pallas_call(
        _tc_out,
        out_shape=jax.ShapeDtypeStruct((NPAD, D_OUT), jnp.float32),
    )(agg2, rin, W2, b2r)

    return out[:N]


# speedup vs baseline: 13.3682x; 1.0555x over previous
import functools

import jax
import jax.numpy as jnp
from jax import lax
from jax.experimental import pallas as pl
from jax.experimental.pallas import tpu as pltpu
from jax.experimental.pallas import tpu_sc as plsc

N = 10000
E = 320000
D_IN = 128
D_HID = 16
D_OUT = 128

NC = 2
NS = 16
NW = NC * NS
CHUNK = 512
CPT = 20
EPT = CPT * CHUNK
EPAD = NW * EPT
NPAD = 10240
RPT = NPAD // NS
DEGW = 8

_mesh = plsc.VectorSubcoreMesh(core_axis_name="c", subcore_axis_name="s")


@functools.partial(
    pl.kernel,
    out_type=jax.ShapeDtypeStruct((NC, 2, NPAD, DEGW), jnp.float32),
    mesh=_mesh,
    scratch_types=[
        pltpu.VMEM((CPT, CHUNK), jnp.int32),
        pltpu.VMEM((CPT, CHUNK), jnp.int32),
        pltpu.VMEM((CHUNK, DEGW), jnp.float32),
        pltpu.VMEM_SHARED((NPAD, DEGW), jnp.float32),
        pltpu.VMEM_SHARED((NPAD, DEGW), jnp.float32),
    ],
    compiler_params=pltpu.CompilerParams(use_tc_tiling_on_sc=False),
)
def _sc_degrees(src_hbm, dst_hbm, ones_hbm, zeros_hbm, out_hbm,
                src_v, dst_v, ones_v, acc_src, acc_dst):
    cid = lax.axis_index("c")
    sid = lax.axis_index("s")
    wid = sid * NC + cid
    pltpu.sync_copy(src_hbm.at[wid], src_v)
    pltpu.sync_copy(dst_hbm.at[wid], dst_v)
    pltpu.sync_copy(ones_hbm, ones_v)
    rows = pl.ds(sid * RPT, RPT)
    pltpu.sync_copy(zeros_hbm.at[rows], acc_src.at[rows])
    pltpu.sync_copy(zeros_hbm.at[rows], acc_dst.at[rows])
    plsc.subcore_barrier()

    def body(j, carry):
        pltpu.sync_copy(ones_v, acc_src.at[src_v.at[j]], add=True)
        pltpu.sync_copy(ones_v, acc_dst.at[dst_v.at[j]], add=True)
        return carry

    lax.fori_loop(0, CPT, body, 0)
    plsc.subcore_barrier()
    pltpu.sync_copy(acc_src.at[rows], out_hbm.at[cid, 0, rows])
    pltpu.sync_copy(acc_dst.at[rows], out_hbm.at[cid, 1, rows])


@functools.partial(
    pl.kernel,
    out_type=jax.ShapeDtypeStruct((NC, NPAD, D_HID), jnp.float32),
    mesh=_mesh,
    scratch_types=[
        pltpu.VMEM((CPT, CHUNK), jnp.int32),
        pltpu.VMEM((CPT, CHUNK), jnp.int32),
        pltpu.VMEM((CHUNK, D_HID), jnp.float32),
        pltpu.VMEM_SHARED((NPAD, D_HID), jnp.float32),
        pltpu.SemaphoreType.DMA,
    ],
    compiler_params=pltpu.CompilerParams(use_tc_tiling_on_sc=False),
)
def _sc_aggregate(t_hbm, src_hbm, dst_hbm, zeros_hbm, out_hbm,
                  src_v, dst_v, msg_v, acc, sem):
    cid = lax.axis_index("c")
    sid = lax.axis_index("s")
    wid = sid * NC + cid
    pltpu.sync_copy(src_hbm.at[wid], src_v)
    pltpu.sync_copy(dst_hbm.at[wid], dst_v)
    rows = pl.ds(sid * RPT, RPT)
    pltpu.sync_copy(zeros_hbm.at[rows], acc.at[rows])
    plsc.subcore_barrier()

    def body(j, carry):
        pltpu.async_copy(t_hbm.at[src_v.at[j]], msg_v, sem).wait()
        pltpu.sync_copy(msg_v, acc.at[dst_v.at[j]], add=True)
        return carry

    lax.fori_loop(0, CPT, body, 0)
    plsc.subcore_barrier()
    pltpu.sync_copy(acc.at[rows], out_hbm.at[cid, rows])


def _tc_prep(nf_ref, w1_ref, degs_ref, t1_ref, rin_ref, rout_ref):
    d_out = degs_ref[0, 0, :, 0:1] + degs_ref[1, 0, :, 0:1]
    d_in = degs_ref[0, 1, :, 0:1] + degs_ref[1, 1, :, 0:1]
    rout = lax.rsqrt(jnp.maximum(d_out, 1.0))
    rin = lax.rsqrt(jnp.maximum(d_in, 1.0))
    p = jnp.dot(nf_ref[...], w1_ref[...], preferred_element_type=jnp.float32)
    t1_ref[...] = p * rout
    rin_ref[...] = jnp.broadcast_to(rin, (NPAD, D_HID))
    rout_ref[...] = jnp.broadcast_to(rout, (NPAD, D_HID))


def _tc_mid(agg_ref, rin_ref, rout_ref, b1_ref, t2_ref):
    s = agg_ref[0] + agg_ref[1]
    h = jnp.maximum(s * rin_ref[...] + b1_ref[...], 0.0)
    t2_ref[...] = h * rout_ref[...]


def _tc_out(agg_ref, rin_ref, w2_ref, b2_ref, out_ref):
    s = (agg_ref[0] + agg_ref[1]) * rin_ref[...]
    out_ref[...] = (
        jnp.dot(s, w2_ref[...], preferred_element_type=jnp.float32)
        + b2_ref[...]
    )


def kernel(node_feat, edge_index, edge_feat, W1, b1, W2, b2):
    del edge_feat

    src = edge_index[0]
    dst = edge_index[1]
    pad = jnp.full((EPAD - E,), N, jnp.int32)
    src_t = jnp.concatenate([src, pad]).reshape(NW, CPT, CHUNK)
    dst_t = jnp.concatenate([dst, pad]).reshape(NW, CPT, CHUNK)

    nf_pad = jnp.zeros((NPAD, D_IN), jnp.float32).at[:N].set(node_feat)
    ones_deg = jnp.ones((CHUNK, DEGW), jnp.float32)
    zeros_deg = jnp.zeros((NPAD, DEGW), jnp.float32)
    zeros_agg = jnp.zeros((NPAD, D_HID), jnp.float32)
    b1r = b1.reshape(1, D_HID)
    b2r = b2.reshape(1, D_OUT)

    degs = _sc_degrees(src_t, dst_t, ones_deg, zeros_deg)

    t1, rin, rout = pl.pallas_call(
        _tc_prep,
        out_shape=(
            jax.ShapeDtypeStruct((NPAD, D_HID), jnp.float32),
            jax.ShapeDtypeStruct((NPAD, D_HID), jnp.float32),
            jax.ShapeDtypeStruct((NPAD, D_HID), jnp.float32),
        ),
    )(nf_pad, W1, degs)

    agg1 = _sc_aggregate(t1, src_t, dst_t, zeros_agg)

    t2 = pl.pallas_call(
        _tc_mid,
        out_shape=jax.ShapeDtypeStruct((NPAD, D_HID), jnp.float32),
    )(agg1, rin, rout, b1r)

    agg2 = _sc_aggregate(t2, src_t, dst_t, zeros_agg)

    out = pl.pallas_call(
        _tc_out,
        out_shape=jax.ShapeDtypeStruct((NPAD, D_OUT), jnp.float32),
    )(agg2, rin, W2, b2r)

    return out[:N]


# speedup vs baseline: 14.3185x; 1.0711x over previous
import functools

import jax
import jax.numpy as jnp
from jax import lax
from jax.experimental import pallas as pl
from jax.experimental.pallas import tpu as pltpu
from jax.experimental.pallas import tpu_sc as plsc

N = 10000
E = 320000
D_IN = 128
D_HID = 16
D_OUT = 128

NC = 2
NS = 16
NW = NC * NS
CHUNK = 512
CPT = 20
EPT = CPT * CHUNK
EPAD = NW * EPT
NPAD = 10240
RPT = NPAD // NS
DEGW = 8

_mesh = plsc.VectorSubcoreMesh(core_axis_name="c", subcore_axis_name="s")


@functools.partial(
    pl.kernel,
    out_type=jax.ShapeDtypeStruct((NC, 2, NPAD, DEGW), jnp.float32),
    mesh=_mesh,
    scratch_types=[
        pltpu.VMEM((CPT, CHUNK), jnp.int32),
        pltpu.VMEM((CPT, CHUNK), jnp.int32),
        pltpu.VMEM((CHUNK, DEGW), jnp.float32),
        pltpu.VMEM_SHARED((NPAD, DEGW), jnp.float32),
        pltpu.VMEM_SHARED((NPAD, DEGW), jnp.float32),
        pltpu.SemaphoreType.DMA,
        pltpu.SemaphoreType.DMA,
    ],
    compiler_params=pltpu.CompilerParams(use_tc_tiling_on_sc=False),
)
def _sc_degrees(src_hbm, dst_hbm, ones_hbm, zeros_hbm, out_hbm,
                src_v, dst_v, ones_v, acc_src, acc_dst, sem_s, sem_d):
    cid = lax.axis_index("c")
    sid = lax.axis_index("s")
    wid = sid * NC + cid
    pltpu.sync_copy(src_hbm.at[wid], src_v)
    pltpu.sync_copy(dst_hbm.at[wid], dst_v)
    pltpu.sync_copy(ones_hbm, ones_v)
    rows = pl.ds(sid * RPT, RPT)
    pltpu.sync_copy(zeros_hbm.at[rows], acc_src.at[rows])
    pltpu.sync_copy(zeros_hbm.at[rows], acc_dst.at[rows])
    plsc.subcore_barrier()

    def body(j, carry):
        pltpu.async_copy(ones_v, acc_src.at[src_v.at[j]], sem_s, add=True)
        pltpu.async_copy(ones_v, acc_dst.at[dst_v.at[j]], sem_d, add=True)

        @pl.when(j > 0)
        def _():
            pltpu.make_async_copy(ones_v, acc_src.at[src_v.at[j]], sem_s).wait()
            pltpu.make_async_copy(ones_v, acc_dst.at[dst_v.at[j]], sem_d).wait()

        return carry

    lax.fori_loop(0, CPT, body, 0)
    pltpu.make_async_copy(ones_v, acc_src.at[src_v.at[0]], sem_s).wait()
    pltpu.make_async_copy(ones_v, acc_dst.at[dst_v.at[0]], sem_d).wait()
    plsc.subcore_barrier()
    pltpu.sync_copy(acc_src.at[rows], out_hbm.at[cid, 0, rows])
    pltpu.sync_copy(acc_dst.at[rows], out_hbm.at[cid, 1, rows])


@functools.partial(
    pl.kernel,
    out_type=jax.ShapeDtypeStruct((NC, NPAD, D_HID), jnp.float32),
    mesh=_mesh,
    scratch_types=[
        pltpu.VMEM((CPT, CHUNK), jnp.int32),
        pltpu.VMEM((CPT, CHUNK), jnp.int32),
        pltpu.VMEM((CHUNK, D_HID), jnp.float32),
        pltpu.VMEM((CHUNK, D_HID), jnp.float32),
        pltpu.VMEM_SHARED((NPAD, D_HID), jnp.float32),
        pltpu.SemaphoreType.DMA,
        pltpu.SemaphoreType.DMA,
        pltpu.SemaphoreType.DMA,
        pltpu.SemaphoreType.DMA,
    ],
    compiler_params=pltpu.CompilerParams(use_tc_tiling_on_sc=False),
)
def _sc_aggregate(t_hbm, src_hbm, dst_hbm, zeros_hbm, out_hbm,
                  src_v, dst_v, msg_a, msg_b, acc, gs_a, gs_b, ss_a, ss_b):
    cid = lax.axis_index("c")
    sid = lax.axis_index("s")
    wid = sid * NC + cid
    pltpu.sync_copy(src_hbm.at[wid], src_v)
    pltpu.sync_copy(dst_hbm.at[wid], dst_v)
    rows = pl.ds(sid * RPT, RPT)
    pltpu.sync_copy(zeros_hbm.at[rows], acc.at[rows])
    plsc.subcore_barrier()

    nk = CPT // 2
    pltpu.async_copy(t_hbm.at[src_v.at[0]], msg_a, gs_a)

    def body(k, carry):
        j0 = 2 * k
        j1 = j0 + 1
        pltpu.make_async_copy(t_hbm.at[src_v.at[j0]], msg_a, gs_a).wait()

        @pl.when(k > 0)
        def _():
            pltpu.make_async_copy(msg_b, acc.at[dst_v.at[j1]], ss_b).wait()

        pltpu.async_copy(t_hbm.at[src_v.at[j1]], msg_b, gs_b)
        pltpu.async_copy(msg_a, acc.at[dst_v.at[j0]], ss_a, add=True)
        pltpu.make_async_copy(t_hbm.at[src_v.at[j1]], msg_b, gs_b).wait()

        @pl.when(k < nk - 1)
        def _():
            pltpu.make_async_copy(msg_a, acc.at[dst_v.at[j0]], ss_a).wait()
            pltpu.async_copy(t_hbm.at[src_v.at[j0 + 2]], msg_a, gs_a)

        pltpu.async_copy(msg_b, acc.at[dst_v.at[j1]], ss_b, add=True)
        return carry

    lax.fori_loop(0, nk, body, 0)
    pltpu.make_async_copy(msg_a, acc.at[dst_v.at[0]], ss_a).wait()
    pltpu.make_async_copy(msg_b, acc.at[dst_v.at[0]], ss_b).wait()
    plsc.subcore_barrier()
    pltpu.sync_copy(acc.at[rows], out_hbm.at[cid, rows])


def _tc_prep(nf_ref, w1_ref, degs_ref, t1_ref, rin_ref, rout_ref):
    d_out = degs_ref[0, 0, :, 0:1] + degs_ref[1, 0, :, 0:1]
    d_in = degs_ref[0, 1, :, 0:1] + degs_ref[1, 1, :, 0:1]
    rout = lax.rsqrt(jnp.maximum(d_out, 1.0))
    rin = lax.rsqrt(jnp.maximum(d_in, 1.0))
    p = jnp.dot(nf_ref[...], w1_ref[...], preferred_element_type=jnp.float32)
    t1_ref[...] = p * rout
    rin_ref[...] = jnp.broadcast_to(rin, (NPAD, D_HID))
    rout_ref[...] = jnp.broadcast_to(rout, (NPAD, D_HID))


def _tc_mid(agg_ref, rin_ref, rout_ref, b1_ref, t2_ref):
    s = agg_ref[0] + agg_ref[1]
    h = jnp.maximum(s * rin_ref[...] + b1_ref[...], 0.0)
    t2_ref[...] = h * rout_ref[...]


def _tc_out(agg_ref, rin_ref, w2_ref, b2_ref, out_ref):
    s = (agg_ref[0] + agg_ref[1]) * rin_ref[...]
    out_ref[...] = (
        jnp.dot(s, w2_ref[...], preferred_element_type=jnp.float32)
        + b2_ref[...]
    )


def kernel(node_feat, edge_index, edge_feat, W1, b1, W2, b2):
    del edge_feat

    src = edge_index[0]
    dst = edge_index[1]
    pad = jnp.full((EPAD - E,), N, jnp.int32)
    src_t = jnp.concatenate([src, pad]).reshape(NW, CPT, CHUNK)
    dst_t = jnp.concatenate([dst, pad]).reshape(NW, CPT, CHUNK)

    nf_pad = jnp.zeros((NPAD, D_IN), jnp.float32).at[:N].set(node_feat)
    ones_deg = jnp.ones((CHUNK, DEGW), jnp.float32)
    zeros_deg = jnp.zeros((NPAD, DEGW), jnp.float32)
    zeros_agg = jnp.zeros((NPAD, D_HID), jnp.float32)
    b1r = b1.reshape(1, D_HID)
    b2r = b2.reshape(1, D_OUT)

    degs = _sc_degrees(src_t, dst_t, ones_deg, zeros_deg)

    t1, rin, rout = pl.pallas_call(
        _tc_prep,
        out_shape=(
            jax.ShapeDtypeStruct((NPAD, D_HID), jnp.float32),
            jax.ShapeDtypeStruct((NPAD, D_HID), jnp.float32),
            jax.ShapeDtypeStruct((NPAD, D_HID), jnp.float32),
        ),
    )(nf_pad, W1, degs)

    agg1 = _sc_aggregate(t1, src_t, dst_t, zeros_agg)

    t2 = pl.pallas_call(
        _tc_mid,
        out_shape=jax.ShapeDtypeStruct((NPAD, D_HID), jnp.float32),
    )(agg1, rin, rout, b1r)

    agg2 = _sc_aggregate(t2, src_t, dst_t, zeros_agg)

    out = pl.pallas_call(
        _tc_out,
        out_shape=jax.ShapeDtypeStruct((NPAD, D_OUT), jnp.float32),
    )(agg2, rin, W2, b2r)

    return out[:N]


# speedup vs baseline: 23.9149x; 1.6702x over previous
import functools

import jax
import jax.numpy as jnp
from jax import lax
from jax.experimental import pallas as pl
from jax.experimental.pallas import tpu as pltpu
from jax.experimental.pallas import tpu_sc as plsc

N = 10000
E = 320000
D_IN = 128
D_HID = 16
D_OUT = 128

NC = 2
NS = 16
NW = NC * NS
CHUNK = 512
CPT = 20
EPT = CPT * CHUNK
EPAD = NW * EPT
NPAD = 10240
RPT = NPAD // NS
RV = RPT // 16
DEGW = 8

_mesh = plsc.VectorSubcoreMesh(core_axis_name="c", subcore_axis_name="s")
_sc_params = pltpu.CompilerParams(
    use_tc_tiling_on_sc=False, needs_layout_passes=False
)


def _rsqrt16(d):
    x = jnp.maximum(d, 1.0)
    yi = jnp.int32(0x5F3759DF) - (plsc.bitcast(x, jnp.int32) >> 1)
    y = plsc.bitcast(yi, jnp.float32)
    y = y * (1.5 - 0.5 * x * y * y)
    y = y * (1.5 - 0.5 * x * y * y)
    y = y * (1.5 - 0.5 * x * y * y)
    return y


def _splat(vec_ref, r):
    return plsc.load_gather(vec_ref, [jnp.full((16,), r, jnp.int32)])


def _col0(cnt8_ref, i):
    ridx = lax.iota(jnp.int32, 16) + i * 16
    return plsc.load_gather(cnt8_ref, [ridx, jnp.zeros((16,), jnp.int32)])


@functools.partial(
    pl.kernel,
    out_type=jax.ShapeDtypeStruct((NC, 2, NPAD, DEGW), jnp.float32),
    mesh=_mesh,
    scratch_types=[
        pltpu.VMEM((CPT, CHUNK), jnp.int32),
        pltpu.VMEM((CPT, CHUNK), jnp.int32),
        pltpu.VMEM((CHUNK, DEGW), jnp.float32),
        pltpu.VMEM_SHARED((NPAD, DEGW), jnp.float32),
        pltpu.VMEM_SHARED((NPAD, DEGW), jnp.float32),
        pltpu.SemaphoreType.DMA,
        pltpu.SemaphoreType.DMA,
    ],
    compiler_params=_sc_params,
)
def _sc_degrees(src_hbm, dst_hbm, ones_hbm, zeros_hbm, out_hbm,
                src_v, dst_v, ones_v, acc_src, acc_dst, sem_s, sem_d):
    cid = lax.axis_index("c")
    sid = lax.axis_index("s")
    wid = sid * NC + cid
    pltpu.sync_copy(src_hbm.at[wid], src_v)
    pltpu.sync_copy(dst_hbm.at[wid], dst_v)
    pltpu.sync_copy(ones_hbm, ones_v)
    rows = pl.ds(sid * RPT, RPT)
    pltpu.sync_copy(zeros_hbm.at[rows], acc_src.at[rows])
    pltpu.sync_copy(zeros_hbm.at[rows], acc_dst.at[rows])
    plsc.subcore_barrier()

    def body(j, carry):
        pltpu.async_copy(ones_v, acc_src.at[src_v.at[j]], sem_s, add=True)
        pltpu.async_copy(ones_v, acc_dst.at[dst_v.at[j]], sem_d, add=True)

        @pl.when(j > 0)
        def _():
            pltpu.make_async_copy(ones_v, acc_src.at[src_v.at[j]], sem_s).wait()
            pltpu.make_async_copy(ones_v, acc_dst.at[dst_v.at[j]], sem_d).wait()

        return carry

    lax.fori_loop(0, CPT, body, 0)
    pltpu.make_async_copy(ones_v, acc_src.at[src_v.at[0]], sem_s).wait()
    pltpu.make_async_copy(ones_v, acc_dst.at[dst_v.at[0]], sem_d).wait()
    plsc.subcore_barrier()
    pltpu.sync_copy(acc_src.at[rows], out_hbm.at[cid, 0, rows])
    pltpu.sync_copy(acc_dst.at[rows], out_hbm.at[cid, 1, rows])


def _agg_main_loop(tbl_sh, acc_sh, src_v, dst_v, msg_a, msg_b,
                   gs_a, gs_b, ss_a, ss_b):
    nk = CPT // 2
    pltpu.async_copy(tbl_sh.at[src_v.at[0]], msg_a, gs_a)

    def body(k, carry):
        j0 = 2 * k
        j1 = j0 + 1
        pltpu.make_async_copy(tbl_sh.at[src_v.at[j0]], msg_a, gs_a).wait()

        @pl.when(k > 0)
        def _():
            pltpu.make_async_copy(msg_b, acc_sh.at[dst_v.at[j1]], ss_b).wait()

        pltpu.async_copy(tbl_sh.at[src_v.at[j1]], msg_b, gs_b)
        pltpu.async_copy(msg_a, acc_sh.at[dst_v.at[j0]], ss_a, add=True)
        pltpu.make_async_copy(tbl_sh.at[src_v.at[j1]], msg_b, gs_b).wait()

        @pl.when(k < nk - 1)
        def _():
            pltpu.make_async_copy(msg_a, acc_sh.at[dst_v.at[j0]], ss_a).wait()
            pltpu.async_copy(tbl_sh.at[src_v.at[j0 + 2]], msg_a, gs_a)

        pltpu.async_copy(msg_b, acc_sh.at[dst_v.at[j1]], ss_b, add=True)
        return carry

    lax.fori_loop(0, nk, body, 0)
    pltpu.make_async_copy(msg_a, acc_sh.at[dst_v.at[0]], ss_a).wait()
    pltpu.make_async_copy(msg_b, acc_sh.at[dst_v.at[0]], ss_b).wait()


_agg_scratch = [
    pltpu.VMEM((CPT, CHUNK), jnp.int32),
    pltpu.VMEM((CPT, CHUNK), jnp.int32),
    pltpu.VMEM((CHUNK, D_HID), jnp.float32),
    pltpu.VMEM((CHUNK, D_HID), jnp.float32),
    pltpu.VMEM((RPT, D_HID), jnp.float32),
    pltpu.VMEM((RPT, D_HID), jnp.float32),
    pltpu.VMEM((RV * 16,), jnp.float32),
    pltpu.VMEM((RV * 16,), jnp.float32),
    pltpu.VMEM((RPT, DEGW), jnp.float32),
    pltpu.VMEM((RPT, DEGW), jnp.float32),
    pltpu.VMEM((16,), jnp.float32),
    pltpu.VMEM_SHARED((NPAD, D_HID), jnp.float32),
    pltpu.VMEM_SHARED((NPAD, D_HID), jnp.float32),
    pltpu.SemaphoreType.DMA,
    pltpu.SemaphoreType.DMA,
    pltpu.SemaphoreType.DMA,
    pltpu.SemaphoreType.DMA,
]


@functools.partial(
    pl.kernel,
    out_type=jax.ShapeDtypeStruct((NC, NPAD, D_HID), jnp.float32),
    mesh=_mesh,
    scratch_types=_agg_scratch,
    compiler_params=_sc_params,
)
def _sc_agg1(p_hbm, cnt_hbm, src_hbm, dst_hbm, zeros_hbm, out_hbm,
             src_v, dst_v, msg_a, msg_b, tab_v, opd_v, rin_v, rout_v,
             ca_v, cb_v, bias_v, tbl_sh, acc_sh, gs_a, gs_b, ss_a, ss_b):
    cid = lax.axis_index("c")
    sid = lax.axis_index("s")
    wid = sid * NC + cid
    pltpu.sync_copy(src_hbm.at[wid], src_v)
    pltpu.sync_copy(dst_hbm.at[wid], dst_v)
    rows = pl.ds(sid * RPT, RPT)
    pltpu.sync_copy(cnt_hbm.at[0, 0, rows], ca_v)
    pltpu.sync_copy(cnt_hbm.at[1, 0, rows], cb_v)
    pltpu.sync_copy(p_hbm.at[rows], tab_v)

    def mk_rout(i, c):
        rout_v[pl.ds(i * 16, 16)] = _rsqrt16(_col0(ca_v, i) + _col0(cb_v, i))
        return c

    lax.fori_loop(0, RV, mk_rout, 0)

    def scale_row(r, c):
        tab_v[r, :] = tab_v[r, :] * _splat(rout_v, r)
        return c

    lax.fori_loop(0, RPT, scale_row, 0)
    pltpu.sync_copy(tab_v, tbl_sh.at[rows])
    pltpu.sync_copy(zeros_hbm.at[rows], acc_sh.at[rows])
    plsc.subcore_barrier()

    _agg_main_loop(tbl_sh, acc_sh, src_v, dst_v, msg_a, msg_b,
                   gs_a, gs_b, ss_a, ss_b)

    plsc.subcore_barrier()
    pltpu.sync_copy(acc_sh.at[rows], out_hbm.at[cid, rows])


@functools.partial(
    pl.kernel,
    out_type=jax.ShapeDtypeStruct((NC, NPAD, D_HID), jnp.float32),
    mesh=_mesh,
    scratch_types=_agg_scratch,
    compiler_params=_sc_params,
)
def _sc_agg2(u1_hbm, cnt_hbm, b1_hbm, src_hbm, dst_hbm, zeros_hbm, out_hbm,
             src_v, dst_v, msg_a, msg_b, tab_v, opd_v, rin_v, rout_v,
             ca_v, cb_v, bias_v, tbl_sh, acc_sh, gs_a, gs_b, ss_a, ss_b):
    cid = lax.axis_index("c")
    sid = lax.axis_index("s")
    wid = sid * NC + cid
    pltpu.sync_copy(src_hbm.at[wid], src_v)
    pltpu.sync_copy(dst_hbm.at[wid], dst_v)
    rows = pl.ds(sid * RPT, RPT)
    pltpu.sync_copy(cnt_hbm.at[0, 0, rows], ca_v)
    pltpu.sync_copy(cnt_hbm.at[1, 0, rows], cb_v)

    def mk_rout(i, c):
        rout_v[pl.ds(i * 16, 16)] = _rsqrt16(_col0(ca_v, i) + _col0(cb_v, i))
        return c

    lax.fori_loop(0, RV, mk_rout, 0)
    pltpu.sync_copy(cnt_hbm.at[0, 1, rows], ca_v)
    pltpu.sync_copy(cnt_hbm.at[1, 1, rows], cb_v)

    def mk_rin(i, c):
        rin_v[pl.ds(i * 16, 16)] = _rsqrt16(_col0(ca_v, i) + _col0(cb_v, i))
        return c

    lax.fori_loop(0, RV, mk_rin, 0)
    pltpu.sync_copy(u1_hbm.at[0, rows], tab_v)
    pltpu.sync_copy(u1_hbm.at[1, rows], opd_v)
    pltpu.sync_copy(b1_hbm, bias_v)

    def mk_t2(r, c):
        s = (tab_v[r, :] + opd_v[r, :]) * _splat(rin_v, r) + bias_v[...]
        tab_v[r, :] = jnp.maximum(s, 0.0) * _splat(rout_v, r)
        return c

    lax.fori_loop(0, RPT, mk_t2, 0)
    pltpu.sync_copy(tab_v, tbl_sh.at[rows])
    pltpu.sync_copy(zeros_hbm.at[rows], acc_sh.at[rows])
    plsc.subcore_barrier()

    _agg_main_loop(tbl_sh, acc_sh, src_v, dst_v, msg_a, msg_b,
                   gs_a, gs_b, ss_a, ss_b)

    plsc.subcore_barrier()
    pltpu.sync_copy(acc_sh.at[rows], tab_v)

    def scale_out(r, c):
        tab_v[r, :] = tab_v[r, :] * _splat(rin_v, r)
        return c

    lax.fori_loop(0, RPT, scale_out, 0)
    pltpu.sync_copy(tab_v, out_hbm.at[cid, rows])


def _tc_in(nf_ref, w1_ref, p_ref):
    p_ref[...] = jnp.dot(nf_ref[...], w1_ref[...],
                         preferred_element_type=jnp.float32)


def _tc_out(s2_ref, w2_ref, b2_ref, out_ref):
    s = s2_ref[0] + s2_ref[1]
    out_ref[...] = (
        jnp.dot(s, w2_ref[...], preferred_element_type=jnp.float32)
        + b2_ref[...]
    )


def kernel(node_feat, edge_index, edge_feat, W1, b1, W2, b2):
    del edge_feat

    src = edge_index[0]
    dst = edge_index[1]
    pad = jnp.full((EPAD - E,), N, jnp.int32)
    src_t = jnp.concatenate([src, pad]).reshape(NW, CPT, CHUNK)
    dst_t = jnp.concatenate([dst, pad]).reshape(NW, CPT, CHUNK)

    nf_pad = jnp.zeros((NPAD, D_IN), jnp.float32).at[:N].set(node_feat)
    ones_deg = jnp.ones((CHUNK, DEGW), jnp.float32)
    zeros_deg = jnp.zeros((NPAD, DEGW), jnp.float32)
    zeros_agg = jnp.zeros((NPAD, D_HID), jnp.float32)
    b2r = b2.reshape(1, D_OUT)

    p = pl.pallas_call(
        _tc_in,
        out_shape=jax.ShapeDtypeStruct((NPAD, D_HID), jnp.float32),
    )(nf_pad, W1)

    cnt = _sc_degrees(src_t, dst_t, ones_deg, zeros_deg)
    u1 = _sc_agg1(p, cnt, src_t, dst_t, zeros_agg)
    s2 = _sc_agg2(u1, cnt, b1, src_t, dst_t, zeros_agg)

    out = pl.pallas_call(
        _tc_out,
        out_shape=jax.ShapeDtypeStruct((NPAD, D_OUT), jnp.float32),
    )(s2, W2, b2r)

    return out[:N]
